# baseline (speedup 1.0000x reference)
import functools

import jax
import jax.numpy as jnp
from jax.experimental import pallas as pl
from jax.experimental.pallas import tpu as pltpu


def _fused(x_ref, w_ref, rep_ref, b_ref, idx_ref, tgt_ref, out_ref, acc_ref,
           *, s_len, n_chunks):
    k = pl.program_id(0)

    @pl.when(k == 0)
    def _init():
        acc_ref[...] = jnp.zeros_like(acc_ref)

    state_chunk = jnp.sum(x_ref[...], axis=1) * (1.0 / s_len)
    acc_ref[...] += jax.lax.dot_general(
        state_chunk, w_ref[...],
        dimension_numbers=(((1,), (1,)), ((), ())),
        preferred_element_type=jnp.float32)

    @pl.when(k == n_chunks - 1)
    def _finish():
        proj = acc_ref[...] + b_ref[...]
        logits_full = jax.lax.dot_general(
            proj, rep_ref[...],
            dimension_numbers=(((1,), (1,)), ((), ())),
            preferred_element_type=jnp.float32)
        n_rows = logits_full.shape[1]
        n_e = idx_ref.shape[1]
        perm = (jax.lax.broadcasted_iota(jnp.int32, (n_rows, n_e), 0)
                == idx_ref[...]).astype(jnp.float32)
        logits = jax.lax.dot_general(
            logits_full, perm,
            dimension_numbers=(((1,), (0,)), ((), ())),
            preferred_element_type=jnp.float32)
        m = jnp.max(logits, axis=1, keepdims=True)
        lse = jnp.log(jnp.sum(jnp.exp(logits - m), axis=1, keepdims=True)) + m
        logp = logits - lse
        onehot = (jax.lax.broadcasted_iota(jnp.int32, (1, n_e), 1)
                  == tgt_ref[...]).astype(jnp.float32)
        picked = jnp.sum(logp * onehot, axis=1, keepdims=True)
        out_ref[...] = jnp.sum(picked, axis=0, keepdims=True) * (
            -0.001 / logits.shape[0])


def kernel(hidden_states, representations, W, b, current_indices,
           current_expert_idx, current_depth):
    B, S, H = hidden_states.shape
    E = current_indices.shape[0]
    C = 512
    n = H // C

    idx2d = current_indices.astype(jnp.int32).reshape(1, E)
    tgt = jnp.asarray(current_expert_idx, jnp.int32).reshape(1, 1)
    b2 = b.reshape(1, H)

    out = pl.pallas_call(
        functools.partial(_fused, s_len=S, n_chunks=n),
        grid=(n,),
        in_specs=[
            pl.BlockSpec((B, S, C), lambda k: (0, 0, k)),
            pl.BlockSpec((H, C), lambda k: (0, k)),
            pl.BlockSpec(representations.shape, lambda k: (0, 0)),
            pl.BlockSpec((1, H), lambda k: (0, 0)),
            pl.BlockSpec((1, E), lambda k: (0, 0)),
            pl.BlockSpec((1, 1), lambda k: (0, 0)),
        ],
        out_specs=pl.BlockSpec((1, 1), lambda k: (0, 0)),
        out_shape=jax.ShapeDtypeStruct((1, 1), jnp.float32),
        scratch_shapes=[pltpu.VMEM((B, H), jnp.float32)],
    )(hidden_states, W, representations, b2, idx2d, tgt)
    return out[0, 0]


# speedup vs baseline: 1.0361x; 1.0361x over previous
import functools

import jax
import jax.numpy as jnp
from jax.experimental import pallas as pl
from jax.experimental.pallas import tpu as pltpu


def _fused(x_ref, w_ref, rep_ref, b_ref, idx_ref, tgt_ref, out_ref, acc_ref,
           *, s_len, n_chunks):
    k = pl.program_id(0)

    @pl.when(k == 0)
    def _init():
        acc_ref[...] = jnp.zeros_like(acc_ref)

    state_chunk = jnp.sum(x_ref[...], axis=1) * (1.0 / s_len)
    acc_ref[...] += jax.lax.dot_general(
        state_chunk, w_ref[...],
        dimension_numbers=(((1,), (1,)), ((), ())),
        preferred_element_type=jnp.float32)

    @pl.when(k == n_chunks - 1)
    def _finish():
        proj = acc_ref[...] + b_ref[...]
        logits_full = jax.lax.dot_general(
            proj, rep_ref[...],
            dimension_numbers=(((1,), (1,)), ((), ())),
            preferred_element_type=jnp.float32)
        n_rows = logits_full.shape[1]
        n_e = idx_ref.shape[1]
        perm = (jax.lax.broadcasted_iota(jnp.int32, (n_rows, n_e), 0)
                == idx_ref[...]).astype(jnp.float32)
        logits = jax.lax.dot_general(
            logits_full, perm,
            dimension_numbers=(((1,), (0,)), ((), ())),
            preferred_element_type=jnp.float32)
        m = jnp.max(logits, axis=1, keepdims=True)
        lse = jnp.log(jnp.sum(jnp.exp(logits - m), axis=1, keepdims=True)) + m
        logp = logits - lse
        onehot = (jax.lax.broadcasted_iota(jnp.int32, (1, n_e), 1)
                  == tgt_ref[...]).astype(jnp.float32)
        picked = jnp.sum(logp * onehot, axis=1, keepdims=True)
        out_ref[...] = jnp.sum(picked, axis=0, keepdims=True) * (
            -0.001 / logits.shape[0])


def kernel(hidden_states, representations, W, b, current_indices,
           current_expert_idx, current_depth):
    B, S, H = hidden_states.shape
    E = current_indices.shape[0]
    C = 256
    n = H // C

    idx2d = current_indices.astype(jnp.int32).reshape(1, E)
    tgt = jnp.asarray(current_expert_idx, jnp.int32).reshape(1, 1)
    b2 = b.reshape(1, H)

    out = pl.pallas_call(
        functools.partial(_fused, s_len=S, n_chunks=n),
        grid=(n,),
        in_specs=[
            pl.BlockSpec((B, S, C), lambda k: (0, 0, k)),
            pl.BlockSpec((H, C), lambda k: (0, k)),
            pl.BlockSpec(representations.shape, lambda k: (0, 0)),
            pl.BlockSpec((1, H), lambda k: (0, 0)),
            pl.BlockSpec((1, E), lambda k: (0, 0)),
            pl.BlockSpec((1, 1), lambda k: (0, 0)),
        ],
        out_specs=pl.BlockSpec((1, 1), lambda k: (0, 0)),
        out_shape=jax.ShapeDtypeStruct((1, 1), jnp.float32),
        scratch_shapes=[pltpu.VMEM((B, H), jnp.float32)],
    )(hidden_states, W, representations, b2, idx2d, tgt)
    return jnp.reshape(out, ())


# speedup vs baseline: 1.0398x; 1.0036x over previous
import functools

import jax
import jax.numpy as jnp
from jax.experimental import pallas as pl
from jax.experimental.pallas import tpu as pltpu


def _fused(x_ref, w_ref, rep_ref, b_ref, idx_ref, tgt_ref, out_ref, acc_ref,
           bias_ref, *, s_len, n_chunks):
    k = pl.program_id(0)

    @pl.when(k == 0)
    def _init():
        acc_ref[...] = jnp.zeros_like(acc_ref)
        bias_ref[...] = jax.lax.dot_general(
            b_ref[...], rep_ref[...],
            dimension_numbers=(((1,), (1,)), ((), ())),
            preferred_element_type=jnp.float32)

    state_chunk = jnp.sum(x_ref[...], axis=1) * (1.0 / s_len)
    g_chunk = jax.lax.dot_general(
        rep_ref[...], w_ref[...],
        dimension_numbers=(((1,), (0,)), ((), ())),
        preferred_element_type=jnp.float32)
    acc_ref[...] += jax.lax.dot_general(
        state_chunk, g_chunk,
        dimension_numbers=(((1,), (1,)), ((), ())),
        preferred_element_type=jnp.float32)

    @pl.when(k == n_chunks - 1)
    def _finish():
        logits_full = acc_ref[...] + bias_ref[...]
        n_rows = logits_full.shape[1]
        n_e = idx_ref.shape[1]
        perm = (jax.lax.broadcasted_iota(jnp.int32, (n_rows, n_e), 0)
                == idx_ref[...]).astype(jnp.float32)
        logits = jax.lax.dot_general(
            logits_full, perm,
            dimension_numbers=(((1,), (0,)), ((), ())),
            preferred_element_type=jnp.float32)
        m = jnp.max(logits, axis=1, keepdims=True)
        lse = jnp.log(jnp.sum(jnp.exp(logits - m), axis=1, keepdims=True)) + m
        logp = logits - lse
        onehot = (jax.lax.broadcasted_iota(jnp.int32, (1, n_e), 1)
                  == tgt_ref[...]).astype(jnp.float32)
        picked = jnp.sum(logp * onehot, axis=1, keepdims=True)
        out_ref[...] = jnp.sum(picked, axis=0, keepdims=True) * (
            -0.001 / logits.shape[0])


def kernel(hidden_states, representations, W, b, current_indices,
           current_expert_idx, current_depth):
    B, S, H = hidden_states.shape
    E = current_indices.shape[0]
    C = 256
    n = H // C

    idx2d = current_indices.astype(jnp.int32).reshape(1, E)
    tgt = jnp.asarray(current_expert_idx, jnp.int32).reshape(1, 1)
    b2 = b.reshape(1, H)

    out = pl.pallas_call(
        functools.partial(_fused, s_len=S, n_chunks=n),
        grid=(n,),
        in_specs=[
            pl.BlockSpec((B, S, C), lambda k: (0, 0, k)),
            pl.BlockSpec((H, C), lambda k: (0, k)),
            pl.BlockSpec(representations.shape, lambda k: (0, 0)),
            pl.BlockSpec((1, H), lambda k: (0, 0)),
            pl.BlockSpec((1, E), lambda k: (0, 0)),
            pl.BlockSpec((1, 1), lambda k: (0, 0)),
        ],
        out_specs=pl.BlockSpec((1, 1), lambda k: (0, 0)),
        out_shape=jax.ShapeDtypeStruct((1, 1), jnp.float32),
        scratch_shapes=[pltpu.VMEM((B, representations.shape[0]), jnp.float32),
                        pltpu.VMEM((1, representations.shape[0]), jnp.float32)],
    )(hidden_states, W, representations, b2, idx2d, tgt)
    return jnp.reshape(out, ())


# speedup vs baseline: 1.0510x; 1.0107x over previous
import functools

import jax
import jax.numpy as jnp
from jax.experimental import pallas as pl
from jax.experimental.pallas import tpu as pltpu


def _fused(x_ref, w_ref, rep_ref, b_ref, idx_ref, tgt_ref, out_ref, acc_ref,
           *, s_len, n_chunks):
    k = pl.program_id(0)

    @pl.when(k == 0)
    def _init():
        acc_ref[...] = jnp.zeros_like(acc_ref)

    state_chunk = jnp.sum(x_ref[...], axis=1) * (1.0 / s_len)
    acc_ref[...] += jax.lax.dot_general(
        state_chunk, w_ref[...],
        dimension_numbers=(((1,), (1,)), ((), ())),
        preferred_element_type=jnp.float32)

    @pl.when(k == n_chunks - 1)
    def _finish():
        proj = acc_ref[...] + b_ref[...]
        logits_full = jax.lax.dot_general(
            proj, rep_ref[...],
            dimension_numbers=(((1,), (1,)), ((), ())),
            preferred_element_type=jnp.float32)
        n_rows = logits_full.shape[1]
        n_e = idx_ref.shape[1]
        perm = (jax.lax.broadcasted_iota(jnp.int32, (n_rows, n_e), 0)
                == idx_ref[...]).astype(jnp.float32)
        logits = jax.lax.dot_general(
            logits_full, perm,
            dimension_numbers=(((1,), (0,)), ((), ())),
            preferred_element_type=jnp.float32)
        m = jnp.max(logits, axis=1, keepdims=True)
        lse = jnp.log(jnp.sum(jnp.exp(logits - m), axis=1, keepdims=True)) + m
        logp = logits - lse
        onehot = (jax.lax.broadcasted_iota(jnp.int32, (1, n_e), 1)
                  == tgt_ref[...]).astype(jnp.float32)
        picked = jnp.sum(logp * onehot, axis=1, keepdims=True)
        out_ref[...] = jnp.sum(picked, axis=0, keepdims=True) * (
            -0.001 / logits.shape[0])


def kernel(hidden_states, representations, W, b, current_indices,
           current_expert_idx, current_depth):
    B, S, H = hidden_states.shape
    E = current_indices.shape[0]
    C = 256
    n = H // C

    idx2d = current_indices.astype(jnp.int32).reshape(1, E)
    tgt = jnp.asarray(current_expert_idx, jnp.int32).reshape(1, 1)
    b2 = b.reshape(1, H)

    out = pl.pallas_call(
        functools.partial(_fused, s_len=S, n_chunks=n),
        grid=(n,),
        in_specs=[
            pl.BlockSpec((B, S, C), lambda k: (0, 0, k)),
            pl.BlockSpec((H, C), lambda k: (0, k)),
            pl.BlockSpec(representations.shape, lambda k: (0, 0)),
            pl.BlockSpec((1, H), lambda k: (0, 0)),
            pl.BlockSpec((1, E), lambda k: (0, 0)),
            pl.BlockSpec((1, 1), lambda k: (0, 0)),
        ],
        out_specs=pl.BlockSpec((1, 1), lambda k: (0, 0)),
        out_shape=jax.ShapeDtypeStruct((1, 1), jnp.float32),
        scratch_shapes=[pltpu.VMEM((B, H), jnp.float32)],
    )(hidden_states, W, representations, b2, idx2d, tgt)
    return jnp.reshape(out, ())
